# baseline (speedup 1.0000x reference)
import functools

import jax
import jax.numpy as jnp
from jax import lax
from jax.experimental import pallas as pl
from jax.experimental.pallas import tpu as pltpu
from jax.experimental.pallas import tpu_sc as plsc

LANES = 16
NBUF = 4


def _emb_kernel(n_rows, hidden, seq_len, rows_per_w, chunk, num_cores):
    n_chunks = rows_per_w // chunk
    n_groups = n_chunks // NBUF
    vecs_per_row = hidden // LANES

    mesh = plsc.VectorSubcoreMesh(core_axis_name="c", subcore_axis_name="s")

    buf_t = pltpu.VMEM((chunk, hidden), jnp.float32)

    @functools.partial(
        pl.kernel,
        mesh=mesh,
        out_type=jax.ShapeDtypeStruct((n_rows, hidden), jnp.float32),
        scratch_types=[
            pltpu.VMEM((rows_per_w,), jnp.int32),
            pltpu.VMEM((seq_len, hidden), jnp.float32),
            buf_t, buf_t, buf_t, buf_t,
            pltpu.SemaphoreType.DMA, pltpu.SemaphoreType.DMA,
            pltpu.SemaphoreType.DMA, pltpu.SemaphoreType.DMA,
            pltpu.SemaphoreType.DMA, pltpu.SemaphoreType.DMA,
            pltpu.SemaphoreType.DMA, pltpu.SemaphoreType.DMA,
        ],
    )
    def emb(
        ids_hbm, tab_hbm, pos_hbm, out_hbm,
        idx_v, pos_v, b0, b1, b2, b3,
        g0, g1, g2, g3, s0, s1, s2, s3,
    ):
        bufs = (b0, b1, b2, b3)
        gsems = (g0, g1, g2, g3)
        ssems = (s0, s1, s2, s3)

        wid = lax.axis_index("s") * num_cores + lax.axis_index("c")
        base = wid * rows_per_w
        pltpu.sync_copy(ids_hbm.at[pl.ds(base, rows_per_w)], idx_v)
        pltpu.sync_copy(pos_hbm, pos_v)

        def start_gather(c, buf, sem):
            pltpu.async_copy(
                tab_hbm.at[idx_v.at[pl.ds(c * chunk, chunk)]], buf, sem
            )

        def wait_gather(buf, sem):
            pltpu.make_async_copy(
                tab_hbm.at[idx_v.at[pl.ds(0, chunk)]], buf, sem
            ).wait()

        def start_store(c, buf, sem):
            pltpu.async_copy(
                buf, out_hbm.at[pl.ds(base + c * chunk, chunk)], sem
            )

        def wait_store(buf, sem):
            pltpu.make_async_copy(
                buf, out_hbm.at[pl.ds(base, chunk)], sem
            ).wait()

        def add_pos(buf, slot):
            def row_body(r, carry):
                pr = slot * chunk + r
                for j in range(vecs_per_row):
                    plsc.addupdate(
                        buf.at[r, pl.ds(j * LANES, LANES)],
                        pos_v[pr, pl.ds(j * LANES, LANES)],
                    )
                return carry

            lax.fori_loop(0, chunk, row_body, 0, unroll=4)

        for k in range(NBUF - 1):
            start_gather(k, bufs[k], gsems[k])

        def group_body(t, carry):
            for k in range(NBUF):
                c = NBUF * t + k
                nk = (k + NBUF - 1) % NBUF

                wait_gather(bufs[k], gsems[k])
                add_pos(bufs[k], k)
                start_store(c, bufs[k], ssems[k])

                if k == 0:
                    @pl.when((c + NBUF - 1 < n_chunks) & (t > 0))
                    def _():
                        wait_store(bufs[nk], ssems[nk])
                else:
                    @pl.when(c + NBUF - 1 < n_chunks)
                    def _():
                        wait_store(bufs[nk], ssems[nk])

                @pl.when(c + NBUF - 1 < n_chunks)
                def _():
                    start_gather(c + NBUF - 1, bufs[nk], gsems[nk])
            return carry

        lax.fori_loop(0, n_groups, group_body, 0, unroll=False)
        for k in range(NBUF):
            wait_store(bufs[k], ssems[k])

    return emb


def kernel(input_ids, tokens_embedding, position_embedding):
    batch, seq_len = input_ids.shape
    vocab, hidden = tokens_embedding.shape
    n_rows = batch * seq_len

    info = plsc.get_sparse_core_info()
    num_workers = info.num_cores * info.num_subcores
    rows_per_w = n_rows // num_workers
    chunk = seq_len // NBUF

    ids_flat = input_ids.reshape(n_rows).astype(jnp.int32)
    emb = _emb_kernel(
        n_rows, hidden, seq_len, rows_per_w, chunk, info.num_cores
    )
    out = emb(ids_flat, tokens_embedding, position_embedding)
    return out.reshape(batch, seq_len, hidden)


# speedup vs baseline: 1.9922x; 1.9922x over previous
import functools

import jax
import jax.numpy as jnp
from jax import lax
from jax.experimental import pallas as pl
from jax.experimental.pallas import tpu as pltpu
from jax.experimental.pallas import tpu_sc as plsc

LANES = 16
NBUF = 4


def _emb_kernel(n_rows, hidden, seq_len, rows_per_w, chunk, num_cores):
    n_chunks = rows_per_w // chunk
    n_groups = n_chunks // NBUF
    vecs_per_row = hidden // LANES

    mesh = plsc.VectorSubcoreMesh(core_axis_name="c", subcore_axis_name="s")

    buf_t = pltpu.VMEM((chunk, hidden), jnp.float32)

    @functools.partial(
        pl.kernel,
        mesh=mesh,
        out_type=jax.ShapeDtypeStruct((n_rows, hidden), jnp.float32),
        scratch_types=[
            pltpu.VMEM((rows_per_w,), jnp.int32),
            pltpu.VMEM((seq_len, hidden), jnp.float32),
            buf_t, buf_t, buf_t, buf_t,
            pltpu.SemaphoreType.DMA, pltpu.SemaphoreType.DMA,
            pltpu.SemaphoreType.DMA, pltpu.SemaphoreType.DMA,
            pltpu.SemaphoreType.DMA, pltpu.SemaphoreType.DMA,
            pltpu.SemaphoreType.DMA, pltpu.SemaphoreType.DMA,
        ],
    )
    def emb(
        ids_hbm, tab_hbm, pos_hbm, out_hbm,
        idx_v, pos_v, b0, b1, b2, b3,
        g0, g1, g2, g3, s0, s1, s2, s3,
    ):
        bufs = (b0, b1, b2, b3)
        gsems = (g0, g1, g2, g3)
        ssems = (s0, s1, s2, s3)

        wid = lax.axis_index("s") * num_cores + lax.axis_index("c")
        base = wid * rows_per_w
        pltpu.sync_copy(ids_hbm.at[pl.ds(base, rows_per_w)], idx_v)
        pltpu.sync_copy(pos_hbm, pos_v)

        def start_gather(c, buf, sem):
            pltpu.async_copy(
                tab_hbm.at[idx_v.at[pl.ds(c * chunk, chunk)]], buf, sem
            )

        def wait_gather(buf, sem):
            pltpu.make_async_copy(
                tab_hbm.at[idx_v.at[pl.ds(0, chunk)]], buf, sem
            ).wait()

        def start_store(c, buf, sem):
            pltpu.async_copy(
                buf, out_hbm.at[pl.ds(base + c * chunk, chunk)], sem
            )

        def wait_store(buf, sem):
            pltpu.make_async_copy(
                buf, out_hbm.at[pl.ds(base, chunk)], sem
            ).wait()

        def add_pos(buf, slot):
            @plsc.parallel_loop(0, chunk, 1, unroll=1)
            def row_body(r):
                pr = slot * chunk + r
                for j in range(vecs_per_row):
                    plsc.addupdate(
                        buf.at[r, pl.ds(j * LANES, LANES)],
                        pos_v[pr, pl.ds(j * LANES, LANES)],
                    )

        for k in range(NBUF - 1):
            start_gather(k, bufs[k], gsems[k])

        def group_body(t, carry):
            for k in range(NBUF):
                c = NBUF * t + k
                nk = (k + NBUF - 1) % NBUF

                wait_gather(bufs[k], gsems[k])
                add_pos(bufs[k], k)
                start_store(c, bufs[k], ssems[k])

                if k == 0:
                    @pl.when((c + NBUF - 1 < n_chunks) & (t > 0))
                    def _():
                        wait_store(bufs[nk], ssems[nk])
                else:
                    @pl.when(c + NBUF - 1 < n_chunks)
                    def _():
                        wait_store(bufs[nk], ssems[nk])

                @pl.when(c + NBUF - 1 < n_chunks)
                def _():
                    start_gather(c + NBUF - 1, bufs[nk], gsems[nk])
            return carry

        lax.fori_loop(0, n_groups, group_body, 0, unroll=False)
        for k in range(NBUF):
            wait_store(bufs[k], ssems[k])

    return emb


def kernel(input_ids, tokens_embedding, position_embedding):
    batch, seq_len = input_ids.shape
    vocab, hidden = tokens_embedding.shape
    n_rows = batch * seq_len

    info = plsc.get_sparse_core_info()
    num_workers = info.num_cores * info.num_subcores
    rows_per_w = n_rows // num_workers
    chunk = seq_len // NBUF

    ids_flat = input_ids.reshape(n_rows).astype(jnp.int32)
    emb = _emb_kernel(
        n_rows, hidden, seq_len, rows_per_w, chunk, info.num_cores
    )
    out = emb(ids_flat, tokens_embedding, position_embedding)
    return out.reshape(batch, seq_len, hidden)
